# baseline (speedup 1.0000x reference)
import functools
import math

import numpy as np
import jax
import jax.numpy as jnp
from jax import lax
from jax.experimental import pallas as pl
from jax.experimental.pallas import tpu as pltpu
from jax.experimental.pallas import tpu_sc as plsc

_D = 64
_DP = 128
_L = 50
_B = 4096
_ROWS = _B * _L
_NC = 2
_NS = 16
_NW = _NC * _NS
_RPW = _ROWS // _NW
_SUB = 80
_SPW = _RPW // _SUB
_CHUNK = 8 * _L
_NCHUNK = _RPW // _CHUNK
_SUBS_PER_CHUNK = _CHUNK // _SUB
_LANES = 16
_VPR = _D // _LANES


_VOCAB = 1000000
_CB = 128
_NBLK = (_VOCAB + _CB - 1) // _CB
_FULL_PER_W = (_NBLK - 5) // _NW
_EXTRA0 = _FULL_PER_W * _NW


def _tr_body(tt_hbm, tail_hbm, out_hbm, ib0, ib1, ob0, ob1, rsem, wsem):
    wid = lax.axis_index("s") * _NC + lax.axis_index("c")
    cbase = wid * _FULL_PER_W

    iot = lax.iota(jnp.int32, _LANES)
    koffs = [(iot + 16 * k) * _D for k in range(_CB // _LANES)]

    ibs = (ib0, ib1)
    obs = (ob0, ob1)

    def read(c, buf):
        pltpu.async_copy(
            tt_hbm.at[:, pl.ds(c * _CB, _CB)], buf, rsem)

    def wait_read(buf):
        pltpu.make_async_copy(
            tt_hbm.at[:, pl.ds(0, _CB)], buf, rsem).wait()

    def transpose(ibuf, obuf, kmax=_CB // _LANES):
        @pl.loop(0, _D)
        def _row(r):
            for k in range(kmax):
                v = ibuf[r, pl.ds(k * _LANES, _LANES)]
                plsc.store_scatter(obuf, [koffs[k] + r], v)

    def write(c, obuf):
        base = pl.multiple_of(c * (_CB * _D), 256)
        pltpu.async_copy(obuf, out_hbm.at[pl.ds(base, _CB * _D)], wsem)

    def wait_write(obuf):
        pltpu.make_async_copy(
            obuf, out_hbm.at[pl.ds(0, _CB * _D)], wsem).wait()

    read(cbase, ibs[0])

    @pl.loop(0, _FULL_PER_W, step=2)
    def _blk(t):
        for ph in range(2):
            ibuf, obuf = ibs[ph], obs[ph]
            wait_read(ibuf)

            @pl.when(t + ph + 1 < _FULL_PER_W)
            def _():
                read(cbase + t + ph + 1, ibs[1 - ph])

            @pl.when(t + ph >= 2)
            def _():
                wait_write(obuf)

            transpose(ibuf, obuf)
            write(cbase + t + ph, obuf)

    wait_write(obs[0])
    wait_write(obs[1])

    @pl.when(wid < _NBLK - 1 - _EXTRA0)
    def _tail():
        c = _EXTRA0 + wid
        read(c, ib0)
        wait_read(ib0)
        transpose(ib0, ob0)
        write(c, ob0)
        wait_write(ob0)

    @pl.when(wid == _NBLK - 1 - _EXTRA0)
    def _last():
        ntail = _VOCAB * _D - (_NBLK - 1) * _CB * _D
        pltpu.sync_copy(tail_hbm, ob0.at[pl.ds(0, ntail)])
        pltpu.sync_copy(ob0.at[pl.ds(0, ntail)],
                        out_hbm.at[pl.ds((_NBLK - 1) * _CB * _D, ntail)])


@functools.lru_cache(maxsize=1)
def _build_transpose():
    mesh = plsc.VectorSubcoreMesh(
        core_axis_name="c", subcore_axis_name="s",
        num_cores=_NC, num_subcores=_NS)
    return functools.partial(
        pl.kernel,
        out_type=jax.ShapeDtypeStruct((_VOCAB * _D,), jnp.float32),
        mesh=mesh,
        scratch_types=[
            pltpu.VMEM((_D, _CB), jnp.float32),
            pltpu.VMEM((_D, _CB), jnp.float32),
            pltpu.VMEM((_CB * _D,), jnp.float32),
            pltpu.VMEM((_CB * _D,), jnp.float32),
            pltpu.SemaphoreType.DMA,
            pltpu.SemaphoreType.DMA,
        ],
        compiler_params=pltpu.CompilerParams(
            use_tc_tiling_on_sc=True, needs_layout_passes=False),
    )(_tr_body)


def _pos_encoding():
    pe = np.zeros((_L, _D), dtype=np.float32)
    pos = np.arange(_L, dtype=np.float32)[:, None]
    div = np.exp(np.arange(0, _D, 2, dtype=np.float32) * -(math.log(10000.0) / _D))
    pe[:, 0::2] = np.sin(pos * div)
    pe[:, 1::2] = np.cos(pos * div)
    return jnp.asarray(pe)


def _emb_body(x_hbm, table_hbm, pe_hbm, out_hbm,
              idx_v, in0, in1, ob0, ob1, pe_v, gsem):
    wid = lax.axis_index("s") * _NC + lax.axis_index("c")
    sub0 = wid * _SPW
    row0 = wid * _RPW

    pltpu.sync_copy(x_hbm.at[pl.ds(sub0, _SPW), :], idx_v)
    pltpu.sync_copy(pe_hbm, pe_v)

    ins = (in0, in1)
    obs = (ob0, ob1)

    def issue(c, buf):
        descs = []
        for j in range(_SUBS_PER_CHUNK):
            descs.append(pltpu.async_copy(
                table_hbm.at[idx_v.at[c * _SUBS_PER_CHUNK + j]],
                buf.at[pl.ds(j * _SUB, _SUB), :],
                gsem))
        return descs

    pending = issue(0, ins[0])
    for c in range(_NCHUNK):
        ibuf = ins[c % 2]
        obuf = obs[c % 2]
        for d in pending:
            d.wait()
        if c + 1 < _NCHUNK:
            pending = issue(c + 1, ins[(c + 1) % 2])

        @pl.loop(0, _L)
        def _add_pe(l, ibuf=ibuf, obuf=obuf):
            for k in range(_VPR):
                pev = pe_v[l, pl.ds(k * _LANES, _LANES)]
                for s in range(_CHUNK // _L):
                    r = s * _L + l
                    off = pl.multiple_of(r * _D + k * _LANES, _LANES)
                    obuf[pl.ds(off, _LANES)] = (
                        ibuf[r, pl.ds(k * _LANES, _LANES)] + pev)

        obase = pl.multiple_of((row0 + c * _CHUNK) * _D, 256)
        pltpu.sync_copy(obuf, out_hbm.at[pl.ds(obase, _CHUNK * _D)])


@functools.lru_cache(maxsize=1)
def _build():
    mesh = plsc.VectorSubcoreMesh(
        core_axis_name="c", subcore_axis_name="s",
        num_cores=_NC, num_subcores=_NS)
    return functools.partial(
        pl.kernel,
        out_type=jax.ShapeDtypeStruct((_ROWS * _D,), jnp.float32),
        mesh=mesh,
        scratch_types=[
            pltpu.VMEM((_SPW, _SUB), jnp.int32),
            pltpu.VMEM((_CHUNK, _D), jnp.float32),
            pltpu.VMEM((_CHUNK, _D), jnp.float32),
            pltpu.VMEM((_CHUNK * _D,), jnp.float32),
            pltpu.VMEM((_CHUNK * _D,), jnp.float32),
            pltpu.VMEM((_L, _D), jnp.float32),
            pltpu.SemaphoreType.DMA,
        ],
        compiler_params=pltpu.CompilerParams(use_tc_tiling_on_sc=False),
    )(_emb_body)


def kernel(x, table):
    xf = x.reshape(_ROWS).astype(jnp.int32).reshape(_NW * _SPW, _SUB)
    tail = table[(_NBLK - 1) * _CB:].reshape(-1)
    t_lin = _build_transpose()(table.T, tail)
    t_rm = t_lin.reshape(_VOCAB, _D)
    out = _build()(xf, t_rm, _pos_encoding())
    return out.reshape(_B, _L, _D)


# speedup vs baseline: 1.7874x; 1.7874x over previous
import functools
import math

import numpy as np
import jax
import jax.numpy as jnp
from jax import lax
from jax.experimental import pallas as pl
from jax.experimental.pallas import tpu as pltpu
from jax.experimental.pallas import tpu_sc as plsc

_D = 64
_L = 50
_B = 4096
_ROWS = _B * _L
_NC = 2
_NS = 16
_NW = _NC * _NS
_RPW = _ROWS // _NW
_SUB = 80
_SPW = _RPW // _SUB
_CHUNK = 16 * _L
_NCHUNK = _RPW // _CHUNK
_SUBS_PER_CHUNK = _CHUNK // _SUB
_LANES = 16
_VPR = _D // _LANES


def _pos_encoding():
    pe = np.zeros((_L, _D), dtype=np.float32)
    pos = np.arange(_L, dtype=np.float32)[:, None]
    div = np.exp(np.arange(0, _D, 2, dtype=np.float32) * -(math.log(10000.0) / _D))
    pe[:, 0::2] = np.sin(pos * div)
    pe[:, 1::2] = np.cos(pos * div)
    return jnp.asarray(pe)


def _emb_body(x_hbm, table_hbm, pe_hbm, out_hbm, idx_v, buf0, buf1, pe_v, gsem):
    wid = lax.axis_index("s") * _NC + lax.axis_index("c")
    sub0 = wid * _SPW
    row0 = wid * _RPW

    pltpu.sync_copy(x_hbm.at[pl.ds(sub0, _SPW), :], idx_v)
    pltpu.sync_copy(pe_hbm, pe_v)

    bufs = (buf0, buf1)

    def issue(c, buf):
        descs = []
        for j in range(_SUBS_PER_CHUNK):
            descs.append(pltpu.async_copy(
                table_hbm.at[idx_v.at[c * _SUBS_PER_CHUNK + j]],
                buf.at[pl.ds(j * _SUB, _SUB), :],
                gsem))
        return descs

    pending = issue(0, bufs[0])
    for c in range(_NCHUNK):
        buf = bufs[c % 2]
        for d in pending:
            d.wait()
        if c + 1 < _NCHUNK:
            pending = issue(c + 1, bufs[(c + 1) % 2])

        @pl.loop(0, _L)
        def _add_pe(l, buf=buf):
            for k in range(_VPR):
                pev = pe_v[l, pl.ds(k * _LANES, _LANES)]
                for s in range(_CHUNK // _L):
                    plsc.addupdate(
                        buf.at[s * _L + l, pl.ds(k * _LANES, _LANES)], pev)

        pltpu.sync_copy(buf, out_hbm.at[pl.ds(row0 + c * _CHUNK, _CHUNK), :])


@functools.lru_cache(maxsize=1)
def _build():
    mesh = plsc.VectorSubcoreMesh(
        core_axis_name="c", subcore_axis_name="s",
        num_cores=_NC, num_subcores=_NS)
    return functools.partial(
        pl.kernel,
        out_type=jax.ShapeDtypeStruct((_ROWS, _D), jnp.float32),
        mesh=mesh,
        scratch_types=[
            pltpu.VMEM((_SPW, _SUB), jnp.int32),
            pltpu.VMEM((_CHUNK, _D), jnp.float32),
            pltpu.VMEM((_CHUNK, _D), jnp.float32),
            pltpu.VMEM((_L, _D), jnp.float32),
            pltpu.SemaphoreType.DMA,
        ],
        compiler_params=pltpu.CompilerParams(use_tc_tiling_on_sc=False),
    )(_emb_body)


def kernel(x, table):
    xf = x.reshape(_ROWS).astype(jnp.int32).reshape(_NW * _SPW, _SUB)
    out = _build()(xf, table, _pos_encoding())
    return out.reshape(_B, _L, _D)
